# Initial kernel scaffold; baseline (speedup 1.0000x reference)
import jax
import jax.numpy as jnp
from jax.experimental import pallas as pl


def kernel(z_t, t, cond, W_ne, b_ne, W_ee, b_ee, W_ue, b_ue, W_e_l, b_e_l, W_n_l, b_n_l, W_no, b_no, W_eo, b_eo, W_h1, b_h1, W_h2, b_h2):
    raise NotImplementedError("write your pallas kernel here")


# speedup vs baseline: 19.1698x; 19.1698x over previous
import math

import jax
import jax.numpy as jnp
from jax.experimental import pallas as pl
from jax.experimental.pallas import tpu as pltpu

B = 32
H = 128
TEMB = 128
COND_DIM = 64
NUM_LAYERS = 3
N_IN, N_HID, N_OUT = 48, 96, 12
N_PER = N_IN + N_HID + N_OUT
E1 = N_IN * N_HID
E2 = N_HID * N_OUT
E_PER = E1 + E2
FLAT = E1 + N_HID + E2


def _dot(a, b):
    return jax.lax.dot_general(a, b, (((1,), (0,)), ((), ())),
                               preferred_element_type=jnp.float32)


def _mpnn_kernel(
    phase_ref,
    u_ref,
    wcol_ref,
    bcol_ref,
    wee_w_ref, wee_b_ref, wee_l1_ref, wee_l2_ref, wee_src_ref, wee_dst_ref,
    wee_t_ref,
    b_ee_ref,
    w_ue_ref,
    b_ue_ref,
    wne0_ref, wne1_ref, wne2_ref,
    b_ne_ref,
    w_e_l_ref,
    be0_ref, be1_ref, be2_ref,
    w_n_l_ref,
    bn0_ref, bn1_ref, bn2_ref,
    w_eo_ref,
    b_eo_ref,
    w_h1_ref,
    b_h1_ref,
    w_h2_ref,
    b_h2_ref,
    out_ref,
):
    f32 = jnp.float32

    phase = phase_ref[0]
    lane = jax.lax.broadcasted_iota(jnp.int32, (1, TEMB), 1)
    temb = jnp.where(lane < TEMB // 2, jnp.cos(phase), jnp.sin(phase))

    h_u = _dot(u_ref[0], w_ue_ref[...]) + b_ue_ref[...]
    common_e = _dot(temb, wee_t_ref[...]) + b_ee_ref[...]

    i1 = jax.lax.broadcasted_iota(jnp.int32, (E1, 1), 0)
    src1 = (i1 % N_IN).astype(f32) * (1.0 / N_PER)
    dst1 = (i1 // N_IN + N_IN).astype(f32) * (1.0 / N_PER)
    he1 = (wcol_ref[0, 0:E1, :] * wee_w_ref[...]
           + bcol_ref[0] * wee_b_ref[...]
           + wee_l1_ref[...]
           + src1 * wee_src_ref[...]
           + dst1 * wee_dst_ref[...]
           + common_e)

    i2 = jax.lax.broadcasted_iota(jnp.int32, (E2, 1), 0)
    src2 = (i2 % N_HID + N_IN).astype(f32) * (1.0 / N_PER)
    dst2 = (i2 // N_HID + N_IN + N_HID).astype(f32) * (1.0 / N_PER)
    he2 = (wcol_ref[0, E1:E_PER, :] * wee_w_ref[...]
           + wee_l2_ref[...]
           + src2 * wee_src_ref[...]
           + dst2 * wee_dst_ref[...]
           + common_e)

    h_in = jnp.broadcast_to(wne0_ref[...] + b_ne_ref[...], (N_IN, H))
    h_hid = jnp.broadcast_to(wne1_ref[...] + b_ne_ref[...], (N_HID, H))
    h_out = jnp.broadcast_to(wne2_ref[...] + b_ne_ref[...], (N_OUT, H))

    be_refs = (be0_ref, be1_ref, be2_ref)
    bn_refs = (bn0_ref, bn1_ref, bn2_ref)

    for l in range(NUM_LAYERS):
        we = w_e_l_ref[l]
        w_src = we[0:H]
        w_dst = we[H:2 * H]
        w_he = we[2 * H:3 * H]
        w_ue2 = we[3 * H:4 * H]

        ce = _dot(h_u, w_ue2) + be_refs[l][...]
        a_in = _dot(h_in, w_src) + ce
        a_hid = _dot(h_hid, w_src)
        b_hid = _dot(h_hid, w_dst)
        b_out = _dot(h_out, w_dst)

        m1 = _dot(he1, w_he)
        m1 = m1 + jnp.broadcast_to(a_in[None], (N_HID, N_IN, H)).reshape(E1, H)
        m1 = m1 + jnp.broadcast_to(b_hid[:, None, :], (N_HID, N_IN, H)).reshape(E1, H)
        he1 = he1 + jnp.maximum(m1, 0.0)

        m2 = _dot(he2, w_he)
        m2 = m2 + jnp.broadcast_to((a_hid + ce)[None], (N_OUT, N_HID, H)).reshape(E2, H)
        m2 = m2 + jnp.broadcast_to(b_out[:, None, :], (N_OUT, N_HID, H)).reshape(E2, H)
        he2 = he2 + jnp.maximum(m2, 0.0)

        agg_hid = he1.reshape(N_HID, N_IN, H).sum(axis=1) * (1.0 / N_IN)
        agg_out = he2.reshape(N_OUT, N_HID, H).sum(axis=1) * (1.0 / N_HID)

        wn = w_n_l_ref[l]
        wn_v = wn[0:H]
        wn_a = wn[H:2 * H]
        wn_u = wn[2 * H:3 * H]
        cn = _dot(h_u, wn_u) + bn_refs[l][...]
        h_in = h_in + jnp.maximum(_dot(h_in, wn_v) + cn, 0.0)
        h_hid = h_hid + jnp.maximum(_dot(h_hid, wn_v) + _dot(agg_hid, wn_a) + cn, 0.0)
        h_out = h_out + jnp.maximum(_dot(h_out, wn_v) + _dot(agg_out, wn_a) + cn, 0.0)

    w_eo = w_eo_ref[...]
    b_eo = b_eo_ref[...]
    w_h1 = w_h1_ref[...]
    b_h1 = b_h1_ref[...]
    w_h2 = w_h2_ref[...]
    b_h2 = b_h2_ref[...]

    eo1 = _dot(he1, w_eo) + b_eo
    hd1 = jnp.maximum(_dot(eo1, w_h1) + b_h1, 0.0)
    out_ref[0, 0:E1, :] = _dot(hd1, w_h2) + b_h2

    eo2 = _dot(he2, w_eo) + b_eo
    hd2 = jnp.maximum(_dot(eo2, w_h1) + b_h1, 0.0)
    out_ref[0, E1:E_PER, :] = _dot(hd2, w_h2) + b_h2


def kernel(z_t, t, cond, W_ne, b_ne, W_ee, b_ee, W_ue, b_ue, W_e_l, b_e_l,
           W_n_l, b_n_l, W_no, b_no, W_eo, b_eo, W_h1, b_h1, W_h2, b_h2):
    del W_no, b_no
    Bn = z_t.shape[0]
    f32 = jnp.float32

    half = TEMB // 2
    freqs = jnp.exp(-math.log(10000.0) * jnp.arange(half, dtype=f32) / half)
    a = t.astype(f32)[:, None] * freqs[None, :]
    phase = jnp.concatenate([a, a], axis=1)

    u = cond[:, 0, :]

    wcol = jnp.concatenate([z_t[:, :E1], z_t[:, E1 + N_HID:]], axis=1)[..., None]
    bcol = jnp.repeat(z_t[:, E1:E1 + N_HID], N_IN, axis=1)[..., None]

    wee_rows = [W_ee[i:i + 1] for i in range(6)]
    wee_t = W_ee[6:]
    wne_rows = [W_ne[i:i + 1] for i in range(3)]
    be_rows = [b_e_l[i:i + 1] for i in range(NUM_LAYERS)]
    bn_rows = [b_n_l[i:i + 1] for i in range(NUM_LAYERS)]

    r2 = lambda v: v.reshape(1, -1)

    full2 = lambda s: pl.BlockSpec(s, lambda b: (0, 0))
    full3 = lambda s: pl.BlockSpec(s, lambda b: (0, 0, 0))

    in_specs = [
        pl.BlockSpec((1, 1, TEMB), lambda b: (b, 0, 0)),
        pl.BlockSpec((1, 1, COND_DIM), lambda b: (b, 0, 0)),
        pl.BlockSpec((1, E_PER, 1), lambda b: (b, 0, 0)),
        pl.BlockSpec((1, E1, 1), lambda b: (b, 0, 0)),
    ] + [full2((1, H))] * 6 + [
        full2((TEMB, H)),
        full2((1, H)),
        full2((COND_DIM, H)),
        full2((1, H)),
        full2((1, H)), full2((1, H)), full2((1, H)),
        full2((1, H)),
        full3((NUM_LAYERS, 4 * H, H)),
        full2((1, H)), full2((1, H)), full2((1, H)),
        full3((NUM_LAYERS, 3 * H, H)),
        full2((1, H)), full2((1, H)), full2((1, H)),
        full2((H, H)),
        full2((1, H)),
        full2((H, H // 2)),
        full2((1, H // 2)),
        full2((H // 2, 1)),
        full2((1, 1)),
    ]

    out = pl.pallas_call(
        _mpnn_kernel,
        grid=(Bn,),
        in_specs=in_specs,
        out_specs=pl.BlockSpec((1, E_PER, 1), lambda b: (b, 0, 0)),
        out_shape=jax.ShapeDtypeStruct((Bn, E_PER, 1), f32),
        compiler_params=pltpu.CompilerParams(
            dimension_semantics=("parallel",)),
    )(
        phase[:, None, :], u[:, None, :], wcol, bcol,
        *wee_rows, wee_t, r2(b_ee),
        W_ue, r2(b_ue),
        *wne_rows, r2(b_ne),
        W_e_l, *be_rows,
        W_n_l, *bn_rows,
        W_eo, r2(b_eo),
        W_h1, r2(b_h1),
        W_h2, r2(b_h2),
    )
    return out.reshape(Bn, E_PER)


# speedup vs baseline: 28.8718x; 1.5061x over previous
import math

import jax
import jax.numpy as jnp
from jax.experimental import pallas as pl
from jax.experimental.pallas import tpu as pltpu

B = 32
H = 128
TEMB = 128
COND_DIM = 64
NUM_LAYERS = 3
N_IN, N_HID, N_OUT = 48, 96, 12
N_PER = N_IN + N_HID + N_OUT
E1 = N_IN * N_HID
E2 = N_HID * N_OUT
E_PER = E1 + E2
FLAT = E1 + N_HID + E2


def _dot(a, b):
    return jax.lax.dot_general(a, b, (((1,), (0,)), ((), ())),
                               preferred_element_type=jnp.float32)


def _tile(x, reps):
    n, h = x.shape
    return jnp.broadcast_to(x[None], (reps, n, h)).reshape(reps * n, h)


def _rep(x, reps):
    n, h = x.shape
    return jnp.broadcast_to(x[:, None, :], (n, reps, h)).reshape(n * reps, h)


def _mpnn_kernel(
    phase_ref,
    u_ref,
    wcol_ref,
    bhid_ref,
    wee_w_ref, wee_b_ref, wee_l1_ref, wee_l2_ref, wee_src_ref, wee_dst_ref,
    wee_t_ref,
    b_ee_ref,
    w_ue_ref,
    b_ue_ref,
    wne0_ref, wne1_ref, wne2_ref,
    b_ne_ref,
    w_e_l_ref,
    be0_ref, be1_ref, be2_ref,
    w_n_l_ref,
    bn0_ref, bn1_ref, bn2_ref,
    w_eo_ref,
    b_eo_ref,
    w_h1_ref,
    b_h1_ref,
    w_h2_ref,
    b_h2_ref,
    out_ref,
):
    f32 = jnp.float32

    phase = phase_ref[0]
    lane = jax.lax.broadcasted_iota(jnp.int32, (1, TEMB), 1)
    temb = jnp.where(lane < TEMB // 2, jnp.cos(phase), jnp.sin(phase))

    h_u = _dot(u_ref[0], w_ue_ref[...]) + b_ue_ref[...]
    common_e = _dot(temb, wee_t_ref[...]) + b_ee_ref[...]

    _iota = lambda n: jax.lax.broadcasted_iota(
        jnp.int32, (n, 1), 0).astype(f32)
    frac48 = _iota(N_IN) * (1.0 / N_PER)
    frac96 = _iota(N_HID) * (1.0 / N_PER)
    frac12 = _iota(N_OUT) * (1.0 / N_PER)
    wee_src = wee_src_ref[...]
    wee_dst = wee_dst_ref[...]

    s1 = wee_l1_ref[...] + common_e + frac48 * wee_src
    r1 = ((frac96 + N_IN / N_PER) * wee_dst
          + bhid_ref[0] * wee_b_ref[...])
    he1 = (wcol_ref[0, 0:E1, :] * wee_w_ref[...]
           + _tile(s1, N_HID) + _rep(r1, N_IN))

    s2 = (wee_l2_ref[...] + common_e
          + (frac96 + N_IN / N_PER) * wee_src)
    r2 = (frac12 + (N_IN + N_HID) / N_PER) * wee_dst
    he2 = (wcol_ref[0, E1:E_PER, :] * wee_w_ref[...]
           + _tile(s2, N_OUT) + _rep(r2, N_HID))

    h_in = jnp.broadcast_to(wne0_ref[...] + b_ne_ref[...], (N_IN, H))
    h_hid = jnp.broadcast_to(wne1_ref[...] + b_ne_ref[...], (N_HID, H))
    h_out = jnp.broadcast_to(wne2_ref[...] + b_ne_ref[...], (N_OUT, H))

    be_refs = (be0_ref, be1_ref, be2_ref)
    bn_refs = (bn0_ref, bn1_ref, bn2_ref)

    for l in range(NUM_LAYERS):
        we = w_e_l_ref[l]
        w_src = we[0:H]
        w_dst = we[H:2 * H]
        w_he = we[2 * H:3 * H]
        w_ue2 = we[3 * H:4 * H]

        ce = _dot(h_u, w_ue2) + be_refs[l][...]
        a_in = _dot(h_in, w_src) + ce
        a_hid = _dot(h_hid, w_src)
        b_hid = _dot(h_hid, w_dst)
        b_out = _dot(h_out, w_dst)

        m1 = _dot(he1, w_he) + _tile(a_in, N_HID) + _rep(b_hid, N_IN)
        he1 = he1 + jnp.maximum(m1, 0.0)

        m2 = _dot(he2, w_he) + _tile(a_hid + ce, N_OUT) + _rep(b_out, N_HID)
        he2 = he2 + jnp.maximum(m2, 0.0)

        agg_hid = he1.reshape(N_HID, N_IN, H).sum(axis=1) * (1.0 / N_IN)
        agg_out = he2.reshape(N_OUT, N_HID, H).sum(axis=1) * (1.0 / N_HID)

        wn = w_n_l_ref[l]
        wn_v = wn[0:H]
        wn_a = wn[H:2 * H]
        wn_u = wn[2 * H:3 * H]
        cn = _dot(h_u, wn_u) + bn_refs[l][...]
        h_in = h_in + jnp.maximum(_dot(h_in, wn_v) + cn, 0.0)
        h_hid = h_hid + jnp.maximum(_dot(h_hid, wn_v) + _dot(agg_hid, wn_a) + cn, 0.0)
        h_out = h_out + jnp.maximum(_dot(h_out, wn_v) + _dot(agg_out, wn_a) + cn, 0.0)

    w_h1 = w_h1_ref[...]
    w_fold = _dot(w_eo_ref[...], w_h1)
    b_fold = _dot(b_eo_ref[...], w_h1) + b_h1_ref[...]
    w_h2 = w_h2_ref[...]
    b_h2 = b_h2_ref[...]

    hd1 = jnp.maximum(_dot(he1, w_fold) + b_fold, 0.0)
    out_ref[0, 0:E1, :] = _dot(hd1, w_h2) + b_h2

    hd2 = jnp.maximum(_dot(he2, w_fold) + b_fold, 0.0)
    out_ref[0, E1:E_PER, :] = _dot(hd2, w_h2) + b_h2


def kernel(z_t, t, cond, W_ne, b_ne, W_ee, b_ee, W_ue, b_ue, W_e_l, b_e_l,
           W_n_l, b_n_l, W_no, b_no, W_eo, b_eo, W_h1, b_h1, W_h2, b_h2):
    del W_no, b_no
    Bn = z_t.shape[0]
    f32 = jnp.float32

    half = TEMB // 2
    freqs = jnp.exp(-math.log(10000.0) * jnp.arange(half, dtype=f32) / half)
    a = t.astype(f32)[:, None] * freqs[None, :]
    phase = jnp.concatenate([a, a], axis=1)

    u = cond[:, 0, :]

    wcol = jnp.concatenate([z_t[:, :E1], z_t[:, E1 + N_HID:]], axis=1)[..., None]
    bhid = z_t[:, E1:E1 + N_HID, None]

    wee_rows = [W_ee[i:i + 1] for i in range(6)]
    wee_t = W_ee[6:]
    wne_rows = [W_ne[i:i + 1] for i in range(3)]
    be_rows = [b_e_l[i:i + 1] for i in range(NUM_LAYERS)]
    bn_rows = [b_n_l[i:i + 1] for i in range(NUM_LAYERS)]

    r2 = lambda v: v.reshape(1, -1)

    full2 = lambda s: pl.BlockSpec(s, lambda b: (0, 0))
    full3 = lambda s: pl.BlockSpec(s, lambda b: (0, 0, 0))

    in_specs = [
        pl.BlockSpec((1, 1, TEMB), lambda b: (b, 0, 0)),
        pl.BlockSpec((1, 1, COND_DIM), lambda b: (b, 0, 0)),
        pl.BlockSpec((1, E_PER, 1), lambda b: (b, 0, 0)),
        pl.BlockSpec((1, N_HID, 1), lambda b: (b, 0, 0)),
    ] + [full2((1, H))] * 6 + [
        full2((TEMB, H)),
        full2((1, H)),
        full2((COND_DIM, H)),
        full2((1, H)),
        full2((1, H)), full2((1, H)), full2((1, H)),
        full2((1, H)),
        full3((NUM_LAYERS, 4 * H, H)),
        full2((1, H)), full2((1, H)), full2((1, H)),
        full3((NUM_LAYERS, 3 * H, H)),
        full2((1, H)), full2((1, H)), full2((1, H)),
        full2((H, H)),
        full2((1, H)),
        full2((H, H // 2)),
        full2((1, H // 2)),
        full2((H // 2, 1)),
        full2((1, 1)),
    ]

    out = pl.pallas_call(
        _mpnn_kernel,
        grid=(Bn,),
        in_specs=in_specs,
        out_specs=pl.BlockSpec((1, E_PER, 1), lambda b: (b, 0, 0)),
        out_shape=jax.ShapeDtypeStruct((Bn, E_PER, 1), f32),
        compiler_params=pltpu.CompilerParams(
            dimension_semantics=("parallel",)),
    )(
        phase[:, None, :], u[:, None, :], wcol, bhid,
        *wee_rows, wee_t, r2(b_ee),
        W_ue, r2(b_ue),
        *wne_rows, r2(b_ne),
        W_e_l, *be_rows,
        W_n_l, *bn_rows,
        W_eo, r2(b_eo),
        W_h1, r2(b_h1),
        W_h2, r2(b_h2),
    )
    return out.reshape(Bn, E_PER)


# speedup vs baseline: 30.8630x; 1.0690x over previous
import math

import jax
import jax.numpy as jnp
from jax.experimental import pallas as pl
from jax.experimental.pallas import tpu as pltpu

H = 128
TEMB = 128
COND_DIM = 64
NUM_LAYERS = 3
N_IN, N_HID, N_OUT = 48, 96, 12
N_PER = N_IN + N_HID + N_OUT
E1 = N_IN * N_HID
E2 = N_HID * N_OUT
E_PER = E1 + E2
FLAT = E1 + N_HID + E2

G = 4


def _dot(a, b):
    return jax.lax.dot_general(a, b, (((1,), (0,)), ((), ())),
                               preferred_element_type=jnp.float32)


def _tile(x, reps):
    n, h = x.shape
    return jnp.broadcast_to(x[None], (reps, n, h)).reshape(reps * n, h)


def _rep(x, reps):
    n, h = x.shape
    return jnp.broadcast_to(x[:, None, :], (n, reps, h)).reshape(n * reps, h)


def _tileg(x, reps):
    n, h = x.shape
    m = n // G
    return jnp.broadcast_to(
        x.reshape(G, 1, m, h), (G, reps, m, h)).reshape(G * reps * m, h)


def _mpnn_kernel(
    phase_ref,
    u_ref,
    wcol_ref,
    bhid_ref,
    wee_w_ref, wee_b_ref, wee_l1_ref, wee_l2_ref, wee_src_ref, wee_dst_ref,
    wee_t_ref,
    b_ee_ref,
    w_ue_ref,
    b_ue_ref,
    wne0_ref, wne1_ref, wne2_ref,
    b_ne_ref,
    w_e_l_ref,
    be0_ref, be1_ref, be2_ref,
    w_n_l_ref,
    bn0_ref, bn1_ref, bn2_ref,
    w_eo_ref,
    b_eo_ref,
    w_h1_ref,
    b_h1_ref,
    w_h2_ref,
    b_h2_ref,
    out_ref,
):
    f32 = jnp.float32

    phase = phase_ref[...].reshape(G, TEMB)
    lane = jax.lax.broadcasted_iota(jnp.int32, (G, TEMB), 1)
    temb = jnp.where(lane < TEMB // 2, jnp.cos(phase), jnp.sin(phase))

    h_u = _dot(u_ref[...].reshape(G, COND_DIM), w_ue_ref[...]) + b_ue_ref[...]
    common_e = _dot(temb, wee_t_ref[...]) + b_ee_ref[...]

    _iota = lambda n: jax.lax.broadcasted_iota(
        jnp.int32, (n, 1), 0).astype(f32)
    frac48 = _iota(N_IN) * (1.0 / N_PER)
    frac96 = _iota(N_HID) * (1.0 / N_PER)
    frac12 = _iota(N_OUT) * (1.0 / N_PER)
    wee_src = wee_src_ref[...]
    wee_dst = wee_dst_ref[...]

    s1 = (_rep(common_e, N_IN)
          + _tile(wee_l1_ref[...] + frac48 * wee_src, G))
    bhid_col = bhid_ref[...].reshape(G * N_HID, 1)
    r1 = (_tile((frac96 + N_IN / N_PER) * wee_dst, G)
          + bhid_col * wee_b_ref[...])
    wcol1 = wcol_ref[:, 0:E1, :].reshape(G * E1, 1)
    he1 = wcol1 * wee_w_ref[...] + _tileg(s1, N_HID) + _rep(r1, N_IN)

    s2 = (_rep(common_e, N_HID)
          + _tile(wee_l2_ref[...] + (frac96 + N_IN / N_PER) * wee_src, G))
    r2 = _tile((frac12 + (N_IN + N_HID) / N_PER) * wee_dst, G)
    wcol2 = wcol_ref[:, E1:E_PER, :].reshape(G * E2, 1)
    he2 = wcol2 * wee_w_ref[...] + _tileg(s2, N_OUT) + _rep(r2, N_HID)

    h_in = jnp.broadcast_to(wne0_ref[...] + b_ne_ref[...], (G * N_IN, H))
    h_hid = jnp.broadcast_to(wne1_ref[...] + b_ne_ref[...], (G * N_HID, H))
    h_out = jnp.broadcast_to(wne2_ref[...] + b_ne_ref[...], (G * N_OUT, H))

    be_refs = (be0_ref, be1_ref, be2_ref)
    bn_refs = (bn0_ref, bn1_ref, bn2_ref)

    for l in range(NUM_LAYERS):
        we = w_e_l_ref[l]
        w_src = we[0:H]
        w_dst = we[H:2 * H]
        w_he = we[2 * H:3 * H]
        w_ue2 = we[3 * H:4 * H]

        ce = _dot(h_u, w_ue2) + be_refs[l][...]
        a_in = _dot(h_in, w_src) + _rep(ce, N_IN)
        a_hid = _dot(h_hid, w_src) + _rep(ce, N_HID)
        b_hid = _dot(h_hid, w_dst)
        b_out = _dot(h_out, w_dst)

        m1 = _dot(he1, w_he) + _tileg(a_in, N_HID) + _rep(b_hid, N_IN)
        he1 = he1 + jnp.maximum(m1, 0.0)

        m2 = _dot(he2, w_he) + _tileg(a_hid, N_OUT) + _rep(b_out, N_HID)
        he2 = he2 + jnp.maximum(m2, 0.0)

        agg_hid = he1.reshape(G * N_HID, N_IN, H).sum(axis=1) * (1.0 / N_IN)
        agg_out = he2.reshape(G * N_OUT, N_HID, H).sum(axis=1) * (1.0 / N_HID)

        wn = w_n_l_ref[l]
        wn_v = wn[0:H]
        wn_a = wn[H:2 * H]
        wn_u = wn[2 * H:3 * H]
        cn = _dot(h_u, wn_u) + bn_refs[l][...]
        h_in = h_in + jnp.maximum(
            _dot(h_in, wn_v) + _rep(cn, N_IN), 0.0)
        h_hid = h_hid + jnp.maximum(
            _dot(h_hid, wn_v) + _dot(agg_hid, wn_a) + _rep(cn, N_HID), 0.0)
        h_out = h_out + jnp.maximum(
            _dot(h_out, wn_v) + _dot(agg_out, wn_a) + _rep(cn, N_OUT), 0.0)

    w_h1 = w_h1_ref[...]
    w_fold = _dot(w_eo_ref[...], w_h1)
    b_fold = _dot(b_eo_ref[...], w_h1) + b_h1_ref[...]
    w_h2 = w_h2_ref[...]
    b_h2 = b_h2_ref[...]

    hd1 = jnp.maximum(_dot(he1, w_fold) + b_fold, 0.0)
    out_ref[:, 0:E1, :] = (_dot(hd1, w_h2) + b_h2).reshape(G, E1, 1)

    hd2 = jnp.maximum(_dot(he2, w_fold) + b_fold, 0.0)
    out_ref[:, E1:E_PER, :] = (_dot(hd2, w_h2) + b_h2).reshape(G, E2, 1)


def kernel(z_t, t, cond, W_ne, b_ne, W_ee, b_ee, W_ue, b_ue, W_e_l, b_e_l,
           W_n_l, b_n_l, W_no, b_no, W_eo, b_eo, W_h1, b_h1, W_h2, b_h2):
    del W_no, b_no
    Bn = z_t.shape[0]
    f32 = jnp.float32

    half = TEMB // 2
    freqs = jnp.exp(-math.log(10000.0) * jnp.arange(half, dtype=f32) / half)
    a = t.astype(f32)[:, None] * freqs[None, :]
    phase = jnp.concatenate([a, a], axis=1)

    u = cond[:, 0, :]

    wcol = jnp.concatenate([z_t[:, :E1], z_t[:, E1 + N_HID:]], axis=1)[..., None]
    bhid = z_t[:, E1:E1 + N_HID, None]

    wee_rows = [W_ee[i:i + 1] for i in range(6)]
    wee_t = W_ee[6:]
    wne_rows = [W_ne[i:i + 1] for i in range(3)]
    be_rows = [b_e_l[i:i + 1] for i in range(NUM_LAYERS)]
    bn_rows = [b_n_l[i:i + 1] for i in range(NUM_LAYERS)]

    r2 = lambda v: v.reshape(1, -1)

    full2 = lambda s: pl.BlockSpec(s, lambda b: (0, 0))
    full3 = lambda s: pl.BlockSpec(s, lambda b: (0, 0, 0))

    in_specs = [
        pl.BlockSpec((G, 1, TEMB), lambda b: (b, 0, 0)),
        pl.BlockSpec((G, 1, COND_DIM), lambda b: (b, 0, 0)),
        pl.BlockSpec((G, E_PER, 1), lambda b: (b, 0, 0)),
        pl.BlockSpec((G, N_HID, 1), lambda b: (b, 0, 0)),
    ] + [full2((1, H))] * 6 + [
        full2((TEMB, H)),
        full2((1, H)),
        full2((COND_DIM, H)),
        full2((1, H)),
        full2((1, H)), full2((1, H)), full2((1, H)),
        full2((1, H)),
        full3((NUM_LAYERS, 4 * H, H)),
        full2((1, H)), full2((1, H)), full2((1, H)),
        full3((NUM_LAYERS, 3 * H, H)),
        full2((1, H)), full2((1, H)), full2((1, H)),
        full2((H, H)),
        full2((1, H)),
        full2((H, H // 2)),
        full2((1, H // 2)),
        full2((H // 2, 1)),
        full2((1, 1)),
    ]

    out = pl.pallas_call(
        _mpnn_kernel,
        grid=(Bn // G,),
        in_specs=in_specs,
        out_specs=pl.BlockSpec((G, E_PER, 1), lambda b: (b, 0, 0)),
        out_shape=jax.ShapeDtypeStruct((Bn, E_PER, 1), f32),
        compiler_params=pltpu.CompilerParams(
            dimension_semantics=("parallel",),
            vmem_limit_bytes=100 * 1024 * 1024),
    )(
        phase[:, None, :], u[:, None, :], wcol, bhid,
        *wee_rows, wee_t, r2(b_ee),
        W_ue, r2(b_ue),
        *wne_rows, r2(b_ne),
        W_e_l, *be_rows,
        W_n_l, *bn_rows,
        W_eo, r2(b_eo),
        W_h1, r2(b_h1),
        W_h2, r2(b_h2),
    )
    return out.reshape(Bn, E_PER)


# speedup vs baseline: 40.0874x; 1.2989x over previous
import math

import jax
import jax.numpy as jnp
from jax.experimental import pallas as pl
from jax.experimental.pallas import tpu as pltpu

H = 128
TEMB = 128
COND_DIM = 64
NUM_LAYERS = 3
N_IN, N_HID, N_OUT = 48, 96, 12
N_PER = N_IN + N_HID + N_OUT
E1 = N_IN * N_HID
E2 = N_HID * N_OUT
E_PER = E1 + E2
FLAT = E1 + N_HID + E2

G = 4


def _dot(a, b):
    return jax.lax.dot_general(a, b, (((1,), (0,)), ((), ())),
                               preferred_element_type=jnp.float32)


def _tile(x, reps):
    n, h = x.shape
    return jnp.broadcast_to(x[None], (reps, n, h)).reshape(reps * n, h)


def _rep(x, reps):
    n, h = x.shape
    return jnp.broadcast_to(x[:, None, :], (n, reps, h)).reshape(n * reps, h)


def _tileg(x, reps):
    n, h = x.shape
    m = n // G
    return jnp.broadcast_to(
        x.reshape(G, 1, m, h), (G, reps, m, h)).reshape(G * reps * m, h)


def _mpnn_kernel(
    phase_ref,
    u_ref,
    wcol_ref,
    bhid_ref,
    wee_w_ref, wee_b_ref, wee_l1_ref, wee_l2_ref, wee_src_ref, wee_dst_ref,
    wee_t_ref,
    b_ee_ref,
    w_ue_ref,
    b_ue_ref,
    wne0_ref, wne1_ref, wne2_ref,
    b_ne_ref,
    w_e_l_ref,
    be0_ref, be1_ref, be2_ref,
    w_n_l_ref,
    bn0_ref, bn1_ref, bn2_ref,
    w_eo_ref,
    b_eo_ref,
    w_h1_ref,
    b_h1_ref,
    w_h2_ref,
    b_h2_ref,
    out_ref,
):
    f32 = jnp.float32

    phase = phase_ref[...].reshape(G, TEMB)
    lane = jax.lax.broadcasted_iota(jnp.int32, (G, TEMB), 1)
    temb = jnp.where(lane < TEMB // 2, jnp.cos(phase), jnp.sin(phase))

    h_u = _dot(u_ref[...].reshape(G, COND_DIM), w_ue_ref[...]) + b_ue_ref[...]
    common_e = _dot(temb, wee_t_ref[...]) + b_ee_ref[...]

    _iota = lambda n: jax.lax.broadcasted_iota(
        jnp.int32, (n, 1), 0).astype(f32)
    frac48 = _iota(N_IN) * (1.0 / N_PER)
    frac96 = _iota(N_HID) * (1.0 / N_PER)
    frac12 = _iota(N_OUT) * (1.0 / N_PER)
    wee_src = wee_src_ref[...]
    wee_dst = wee_dst_ref[...]

    s1 = (_rep(common_e, N_IN)
          + _tile(wee_l1_ref[...] + frac48 * wee_src, G))
    bhid_col = bhid_ref[...].reshape(G * N_HID, 1)
    r1 = (_tile((frac96 + N_IN / N_PER) * wee_dst, G)
          + bhid_col * wee_b_ref[...])
    wrow = wcol_ref[...].reshape(1, G * E_PER)
    w3 = jnp.transpose(wrow, (1, 0)).reshape(G, E_PER, 1)
    wcol1 = w3[:, 0:E1, :].reshape(G * E1, 1)
    he1 = wcol1 * wee_w_ref[...] + _tileg(s1, N_HID) + _rep(r1, N_IN)

    s2 = (_rep(common_e, N_HID)
          + _tile(wee_l2_ref[...] + (frac96 + N_IN / N_PER) * wee_src, G))
    r2 = _tile((frac12 + (N_IN + N_HID) / N_PER) * wee_dst, G)
    wcol2 = w3[:, E1:E_PER, :].reshape(G * E2, 1)
    he2 = wcol2 * wee_w_ref[...] + _tileg(s2, N_OUT) + _rep(r2, N_HID)

    h_in = jnp.broadcast_to(wne0_ref[...] + b_ne_ref[...], (G * N_IN, H))
    h_hid = jnp.broadcast_to(wne1_ref[...] + b_ne_ref[...], (G * N_HID, H))
    h_out = jnp.broadcast_to(wne2_ref[...] + b_ne_ref[...], (G * N_OUT, H))

    be_refs = (be0_ref, be1_ref, be2_ref)
    bn_refs = (bn0_ref, bn1_ref, bn2_ref)

    for l in range(NUM_LAYERS):
        we = w_e_l_ref[l]
        w_src = we[0:H]
        w_dst = we[H:2 * H]
        w_he = we[2 * H:3 * H]
        w_ue2 = we[3 * H:4 * H]

        ce = _dot(h_u, w_ue2) + be_refs[l][...]
        a_in = _dot(h_in, w_src) + _rep(ce, N_IN)
        a_hid = _dot(h_hid, w_src) + _rep(ce, N_HID)
        b_hid = _dot(h_hid, w_dst)
        b_out = _dot(h_out, w_dst)

        m1 = _dot(he1, w_he) + _tileg(a_in, N_HID) + _rep(b_hid, N_IN)
        he1 = he1 + jnp.maximum(m1, 0.0)

        m2 = _dot(he2, w_he) + _tileg(a_hid, N_OUT) + _rep(b_out, N_HID)
        he2 = he2 + jnp.maximum(m2, 0.0)

        agg_hid = he1.reshape(G * N_HID, N_IN, H).sum(axis=1) * (1.0 / N_IN)
        agg_out = he2.reshape(G * N_OUT, N_HID, H).sum(axis=1) * (1.0 / N_HID)

        wn = w_n_l_ref[l]
        wn_v = wn[0:H]
        wn_a = wn[H:2 * H]
        wn_u = wn[2 * H:3 * H]
        cn = _dot(h_u, wn_u) + bn_refs[l][...]
        h_in = h_in + jnp.maximum(
            _dot(h_in, wn_v) + _rep(cn, N_IN), 0.0)
        h_hid = h_hid + jnp.maximum(
            _dot(h_hid, wn_v) + _dot(agg_hid, wn_a) + _rep(cn, N_HID), 0.0)
        h_out = h_out + jnp.maximum(
            _dot(h_out, wn_v) + _dot(agg_out, wn_a) + _rep(cn, N_OUT), 0.0)

    w_h1 = w_h1_ref[...]
    w_fold = _dot(w_eo_ref[...], w_h1)
    b_fold = _dot(b_eo_ref[...], w_h1) + b_h1_ref[...]
    w_h2 = w_h2_ref[...]
    b_h2 = b_h2_ref[...]

    hd1 = jnp.maximum(_dot(he1, w_fold) + b_fold, 0.0)
    y1 = (_dot(hd1, w_h2) + b_h2).reshape(G, E1, 1)
    hd2 = jnp.maximum(_dot(he2, w_fold) + b_fold, 0.0)
    y2 = (_dot(hd2, w_h2) + b_h2).reshape(G, E2, 1)
    ycol = jnp.concatenate([y1, y2], axis=1).reshape(G * E_PER, 1)
    out_ref[...] = ycol.reshape(G, E_PER // 128, 128)


def kernel(z_t, t, cond, W_ne, b_ne, W_ee, b_ee, W_ue, b_ue, W_e_l, b_e_l,
           W_n_l, b_n_l, W_no, b_no, W_eo, b_eo, W_h1, b_h1, W_h2, b_h2):
    del W_no, b_no
    Bn = z_t.shape[0]
    f32 = jnp.float32

    half = TEMB // 2
    freqs = jnp.exp(-math.log(10000.0) * jnp.arange(half, dtype=f32) / half)
    a = t.astype(f32)[:, None] * freqs[None, :]
    phase = jnp.concatenate([a, a], axis=1)

    u = cond[:, 0, :]

    wcol = jnp.concatenate(
        [z_t[:, :E1], z_t[:, E1 + N_HID:]], axis=1).reshape(Bn, E_PER // 128, 128)
    bhid = z_t[:, E1:E1 + N_HID, None]

    wee_rows = [W_ee[i:i + 1] for i in range(6)]
    wee_t = W_ee[6:]
    wne_rows = [W_ne[i:i + 1] for i in range(3)]
    be_rows = [b_e_l[i:i + 1] for i in range(NUM_LAYERS)]
    bn_rows = [b_n_l[i:i + 1] for i in range(NUM_LAYERS)]

    r2 = lambda v: v.reshape(1, -1)

    full2 = lambda s: pl.BlockSpec(s, lambda b: (0, 0))
    full3 = lambda s: pl.BlockSpec(s, lambda b: (0, 0, 0))

    in_specs = [
        pl.BlockSpec((G, 1, TEMB), lambda b: (b, 0, 0)),
        pl.BlockSpec((G, 1, COND_DIM), lambda b: (b, 0, 0)),
        pl.BlockSpec((G, E_PER // 128, 128), lambda b: (b, 0, 0)),
        pl.BlockSpec((G, N_HID, 1), lambda b: (b, 0, 0)),
    ] + [full2((1, H))] * 6 + [
        full2((TEMB, H)),
        full2((1, H)),
        full2((COND_DIM, H)),
        full2((1, H)),
        full2((1, H)), full2((1, H)), full2((1, H)),
        full2((1, H)),
        full3((NUM_LAYERS, 4 * H, H)),
        full2((1, H)), full2((1, H)), full2((1, H)),
        full3((NUM_LAYERS, 3 * H, H)),
        full2((1, H)), full2((1, H)), full2((1, H)),
        full2((H, H)),
        full2((1, H)),
        full2((H, H // 2)),
        full2((1, H // 2)),
        full2((H // 2, 1)),
        full2((1, 1)),
    ]

    out = pl.pallas_call(
        _mpnn_kernel,
        grid=(Bn // G,),
        in_specs=in_specs,
        out_specs=pl.BlockSpec((G, E_PER // 128, 128), lambda b: (b, 0, 0)),
        out_shape=jax.ShapeDtypeStruct((Bn, E_PER // 128, 128), f32),
        compiler_params=pltpu.CompilerParams(
            dimension_semantics=("parallel",),
            vmem_limit_bytes=100 * 1024 * 1024),
    )(
        phase[:, None, :], u[:, None, :], wcol, bhid,
        *wee_rows, wee_t, r2(b_ee),
        W_ue, r2(b_ue),
        *wne_rows, r2(b_ne),
        W_e_l, *be_rows,
        W_n_l, *bn_rows,
        W_eo, r2(b_eo),
        W_h1, r2(b_h1),
        W_h2, r2(b_h2),
    )
    return out.reshape(Bn, E_PER)
